# Initial kernel scaffold; baseline (speedup 1.0000x reference)
import jax
import jax.numpy as jnp
from jax.experimental import pallas as pl


def kernel(x, edge_index, W1l, b1, W1r, W2l, b2, W2r, W3l, b3, W3r):
    raise NotImplementedError("write your pallas kernel here")


# speedup vs baseline: 4.6411x; 4.6411x over previous
import functools

import jax
import jax.numpy as jnp
from jax import lax
from jax.experimental import pallas as pl
from jax.experimental.pallas import tpu as pltpu
from jax.experimental.pallas import tpu_sc as plsc

NC = 2
NS = 16
NW = NC * NS
CH = 80
RN = 80


def _rr_rows(s, n, copy_one):
    n_chunks = n // RN
    iters = (n_chunks + NS - 1) // NS

    def body(t, carry):
        cid = s + t * NS

        @pl.when(cid < n_chunks)
        def _():
            copy_one(cid * RN)

        return carry

    lax.fori_loop(0, iters, body, 0)


def _segsum_sc(x, src, dst, zeros):
    n, d = x.shape
    e = src.shape[0]
    e_per_w = e // NW
    n_chunks = e_per_w // CH
    n_per_s = n // NS
    mesh = plsc.VectorSubcoreMesh(core_axis_name="c", subcore_axis_name="s")

    @functools.partial(
        pl.kernel,
        out_type=jax.ShapeDtypeStruct((NC, n, d), jnp.float32),
        mesh=mesh,
        scratch_types=[
            pltpu.VMEM((CH,), jnp.int32),
            pltpu.VMEM((CH,), jnp.int32),
            pltpu.VMEM((CH, d), jnp.float32),
            pltpu.VMEM_SHARED((n, d), jnp.float32),
            pltpu.SemaphoreType.DMA,
        ],
    )
    def k(x_hbm, src_hbm, dst_hbm, zeros_hbm, out_hbm, sidx, didx, rows, acc, sem):
        c = lax.axis_index("c")
        s = lax.axis_index("s")
        base = (c * NS + s) * e_per_w
        _rr_rows(s, n, lambda r0: pltpu.sync_copy(
            zeros_hbm.at[pl.ds(r0, RN)], acc.at[pl.ds(r0, RN)]))
        plsc.subcore_barrier()

        def body(j, carry):
            off = base + j * CH
            pltpu.sync_copy(src_hbm.at[pl.ds(off, CH)], sidx)
            pltpu.sync_copy(dst_hbm.at[pl.ds(off, CH)], didx)
            pltpu.async_copy(x_hbm.at[sidx], rows, sem).wait()
            pltpu.sync_copy(rows, acc.at[didx], add=True)
            return carry

        lax.fori_loop(0, n_chunks, body, 0)
        plsc.subcore_barrier()
        _rr_rows(s, n, lambda r0: pltpu.sync_copy(
            acc.at[pl.ds(r0, RN)], out_hbm.at[c, pl.ds(r0, RN)]))

    return k(x, src, dst, zeros)


def _count_sc(dst, zeros, n, d):
    e = dst.shape[0]
    e_per_w = e // NW
    n_chunks = e_per_w // CH
    mesh = plsc.VectorSubcoreMesh(core_axis_name="c", subcore_axis_name="s")

    @functools.partial(
        pl.kernel,
        out_type=jax.ShapeDtypeStruct((NC, n, d), jnp.float32),
        mesh=mesh,
        scratch_types=[
            pltpu.VMEM((CH,), jnp.int32),
            pltpu.VMEM((CH, d), jnp.float32),
            pltpu.VMEM_SHARED((n, d), jnp.float32),
        ],
    )
    def k(dst_hbm, zeros_hbm, out_hbm, didx, ones_v, acc):
        c = lax.axis_index("c")
        s = lax.axis_index("s")
        base = (c * NS + s) * e_per_w

        def fill(i, carry):
            for kk in range(d // 16):
                ones_v[i, pl.ds(kk * 16, 16)] = jnp.ones((16,), jnp.float32)
            return carry

        lax.fori_loop(0, CH, fill, 0)
        _rr_rows(s, n, lambda r0: pltpu.sync_copy(
            zeros_hbm.at[pl.ds(r0, RN)], acc.at[pl.ds(r0, RN)]))
        plsc.subcore_barrier()

        def body(j, carry):
            off = base + j * CH
            pltpu.sync_copy(dst_hbm.at[pl.ds(off, CH)], didx)
            pltpu.sync_copy(ones_v, acc.at[didx], add=True)
            return carry

        lax.fori_loop(0, n_chunks, body, 0)
        plsc.subcore_barrier()
        _rr_rows(s, n, lambda r0: pltpu.sync_copy(
            acc.at[pl.ds(r0, RN)], out_hbm.at[c, pl.ds(r0, RN)]))

    return k(dst, zeros)


def _dense_tc(aggp, cntp, h, wl_t, bl, wr_t, relu):
    n, d = h.shape
    bn = 1000

    def body(ap_ref, cp_ref, h_ref, wl_ref, bl_ref, wr_ref, o_ref):
        agg = ap_ref[0] + ap_ref[1]
        cnt = cp_ref[0, :, 0:1] + cp_ref[1, :, 0:1]
        mean = agg / jnp.maximum(cnt, 1.0)
        y = (jnp.dot(mean, wl_ref[...], preferred_element_type=jnp.float32,
                     precision=lax.Precision.HIGHEST)
             + bl_ref[...]
             + jnp.dot(h_ref[...], wr_ref[...], preferred_element_type=jnp.float32,
                       precision=lax.Precision.HIGHEST))
        if relu:
            y = jnp.maximum(y, 0.0)
        o_ref[...] = y

    return pl.pallas_call(
        body,
        out_shape=jax.ShapeDtypeStruct((n, d), jnp.float32),
        grid=(n // bn,),
        in_specs=[
            pl.BlockSpec((NC, bn, d), lambda i: (0, i, 0)),
            pl.BlockSpec((NC, bn, d), lambda i: (0, i, 0)),
            pl.BlockSpec((bn, d), lambda i: (i, 0)),
            pl.BlockSpec((d, d), lambda i: (0, 0)),
            pl.BlockSpec((1, d), lambda i: (0, 0)),
            pl.BlockSpec((d, d), lambda i: (0, 0)),
        ],
        out_specs=pl.BlockSpec((bn, d), lambda i: (i, 0)),
    )(aggp, cntp, h, wl_t, bl, wr_t)


def kernel(x, edge_index, W1l, b1, W1r, W2l, b2, W2r, W3l, b3, W3r):
    n, d = x.shape
    ei = edge_index.astype(jnp.int32)
    src, dst = ei[0], ei[1]
    zeros = jnp.zeros((n, d), jnp.float32)
    cntp = _count_sc(dst, zeros, n, d)

    h = x
    for wl, bl, wr, relu in (
        (W1l, b1, W1r, True),
        (W2l, b2, W2r, True),
        (W3l, b3, W3r, False),
    ):
        aggp = _segsum_sc(h, src, dst, zeros)
        h = _dense_tc(aggp, cntp, h, wl.T, bl.reshape(1, d), wr.T, relu)
    return h


# speedup vs baseline: 8.0680x; 1.7384x over previous
import functools

import jax
import jax.numpy as jnp
from jax import lax
from jax.experimental import pallas as pl
from jax.experimental.pallas import tpu as pltpu
from jax.experimental.pallas import tpu_sc as plsc

NC = 2
NS = 16
NW = NC * NS
CH = 80
RN = 80
SUP = 25


def _rr_rows(s, n, copy_one):
    n_chunks = n // RN
    iters = (n_chunks + NS - 1) // NS

    def body(t, carry):
        cid = s + t * NS

        @pl.when(cid < n_chunks)
        def _():
            copy_one(cid * RN)

        return carry

    lax.fori_loop(0, iters, body, 0)


def _segsum_sc(x, src, dst, zeros):
    n, d = x.shape
    e = src.shape[0]
    e_per_w = e // NW
    n_chunks = e_per_w // CH
    mesh = plsc.VectorSubcoreMesh(core_axis_name="c", subcore_axis_name="s")

    @functools.partial(
        pl.kernel,
        out_type=jax.ShapeDtypeStruct((NC, n, d), jnp.float32),
        mesh=mesh,
        scratch_types=[
            pltpu.VMEM((SUP * CH,), jnp.int32),
            pltpu.VMEM((SUP * CH,), jnp.int32),
            pltpu.VMEM((CH, d), jnp.float32),
            pltpu.VMEM((CH, d), jnp.float32),
            pltpu.VMEM_SHARED((n, d), jnp.float32),
            pltpu.SemaphoreType.DMA,
            pltpu.SemaphoreType.DMA,
        ],
    )
    def k(x_hbm, src_hbm, dst_hbm, zeros_hbm, out_hbm,
          sidx, didx, rows0, rows1, acc, g0, g1):
        c = lax.axis_index("c")
        s = lax.axis_index("s")
        w = c * NS + s
        _rr_rows(s, n, lambda r0: pltpu.sync_copy(
            zeros_hbm.at[pl.ds(r0, RN)], acc.at[pl.ds(r0, RN)]))
        plsc.subcore_barrier()

        def start_g(j, rows, sem):
            pltpu.async_copy(x_hbm.at[sidx.at[pl.ds(j * CH, CH)]], rows, sem)

        def wait_g(rows, sem):
            pltpu.make_async_copy(zeros_hbm.at[pl.ds(0, CH)], rows, sem).wait()

        def scatter(j, rows):
            pltpu.sync_copy(rows, acc.at[didx.at[pl.ds(j * CH, CH)]], add=True)

        def super_body(u, carry):
            off = w * e_per_w + u * (SUP * CH)
            pltpu.sync_copy(src_hbm.at[pl.ds(off, SUP * CH)], sidx)
            pltpu.sync_copy(dst_hbm.at[pl.ds(off, SUP * CH)], didx)
            start_g(0, rows0, g0)

            def body(t, carry2):
                a = 2 * t
                wait_g(rows0, g0)
                start_g(a + 1, rows1, g1)
                scatter(a, rows0)
                wait_g(rows1, g1)

                @pl.when(a + 2 < SUP)
                def _():
                    start_g(a + 2, rows0, g0)

                scatter(a + 1, rows1)
                return carry2

            lax.fori_loop(0, SUP // 2, body, 0)
            if SUP % 2:
                wait_g(rows0, g0)
                scatter(SUP - 1, rows0)
            return carry

        lax.fori_loop(0, n_chunks // SUP, super_body, 0)
        plsc.subcore_barrier()
        _rr_rows(s, n, lambda r0: pltpu.sync_copy(
            acc.at[pl.ds(r0, RN)], out_hbm.at[c, pl.ds(r0, RN)]))

    return k(x, src, dst, zeros)


def _count_sc(dst, zeros, n, d):
    e = dst.shape[0]
    e_per_w = e // NW
    n_chunks = e_per_w // CH
    depth = 2
    mesh = plsc.VectorSubcoreMesh(core_axis_name="c", subcore_axis_name="s")

    @functools.partial(
        pl.kernel,
        out_type=jax.ShapeDtypeStruct((NC, n, d), jnp.float32),
        mesh=mesh,
        scratch_types=[
            pltpu.VMEM((SUP * CH,), jnp.int32),
            pltpu.VMEM((CH, d), jnp.float32),
            pltpu.VMEM_SHARED((n, d), jnp.float32),
            pltpu.SemaphoreType.DMA,
        ],
    )
    def k(dst_hbm, zeros_hbm, out_hbm, didx, ones_v, acc, ssem):
        c = lax.axis_index("c")
        s = lax.axis_index("s")
        w = c * NS + s

        def fill(i, carry):
            for kk in range(d // 16):
                ones_v[i, pl.ds(kk * 16, 16)] = jnp.ones((16,), jnp.float32)
            return carry

        lax.fori_loop(0, CH, fill, 0)
        _rr_rows(s, n, lambda r0: pltpu.sync_copy(
            zeros_hbm.at[pl.ds(r0, RN)], acc.at[pl.ds(r0, RN)]))
        plsc.subcore_barrier()

        def wait_one():
            pltpu.make_async_copy(zeros_hbm.at[pl.ds(0, CH)], ones_v, ssem).wait()

        def super_body(u, carry):
            off = w * e_per_w + u * (SUP * CH)
            pltpu.sync_copy(dst_hbm.at[pl.ds(off, SUP * CH)], didx)

            def body(j, carry2):
                pltpu.async_copy(ones_v, acc.at[didx.at[pl.ds(j * CH, CH)]],
                                 ssem, add=True)

                @pl.when(j >= depth)
                def _():
                    wait_one()

                return carry2

            lax.fori_loop(0, SUP, body, 0)
            for _ in range(min(depth, SUP)):
                wait_one()
            return carry

        lax.fori_loop(0, n_chunks // SUP, super_body, 0)
        plsc.subcore_barrier()
        _rr_rows(s, n, lambda r0: pltpu.sync_copy(
            acc.at[pl.ds(r0, RN)], out_hbm.at[c, pl.ds(r0, RN)]))

    return k(dst, zeros)


def _dense_tc(aggp, cntp, h, wl_t, bl, wr_t, relu):
    n, d = h.shape
    bn = 1000

    def body(ap_ref, cp_ref, h_ref, wl_ref, bl_ref, wr_ref, o_ref):
        agg = ap_ref[0] + ap_ref[1]
        cnt = cp_ref[0, :, 0:1] + cp_ref[1, :, 0:1]
        mean = agg / jnp.maximum(cnt, 1.0)
        y = (jnp.dot(mean, wl_ref[...], preferred_element_type=jnp.float32,
                     precision=lax.Precision.HIGHEST)
             + bl_ref[...]
             + jnp.dot(h_ref[...], wr_ref[...], preferred_element_type=jnp.float32,
                       precision=lax.Precision.HIGHEST))
        if relu:
            y = jnp.maximum(y, 0.0)
        o_ref[...] = y

    return pl.pallas_call(
        body,
        out_shape=jax.ShapeDtypeStruct((n, d), jnp.float32),
        grid=(n // bn,),
        in_specs=[
            pl.BlockSpec((NC, bn, d), lambda i: (0, i, 0)),
            pl.BlockSpec((NC, bn, d), lambda i: (0, i, 0)),
            pl.BlockSpec((bn, d), lambda i: (i, 0)),
            pl.BlockSpec((d, d), lambda i: (0, 0)),
            pl.BlockSpec((1, d), lambda i: (0, 0)),
            pl.BlockSpec((d, d), lambda i: (0, 0)),
        ],
        out_specs=pl.BlockSpec((bn, d), lambda i: (i, 0)),
    )(aggp, cntp, h, wl_t, bl, wr_t)


def kernel(x, edge_index, W1l, b1, W1r, W2l, b2, W2r, W3l, b3, W3r):
    n, d = x.shape
    ei = edge_index.astype(jnp.int32)
    src, dst = ei[0], ei[1]
    zeros = jnp.zeros((n, d), jnp.float32)

    cntp = _count_sc(dst, zeros, n, d)

    h = x
    for wl, bl, wr, relu in (
        (W1l, b1, W1r, True),
        (W2l, b2, W2r, True),
        (W3l, b3, W3r, False),
    ):
        aggp = _segsum_sc(h, src, dst, zeros)
        h = _dense_tc(aggp, cntp, h, wl.T, bl.reshape(1, d), wr.T, relu)
    return h


# speedup vs baseline: 9.5381x; 1.1822x over previous
import functools

import jax
import jax.numpy as jnp
from jax import lax
from jax.experimental import pallas as pl
from jax.experimental.pallas import tpu as pltpu
from jax.experimental.pallas import tpu_sc as plsc

NC = 2
NS = 16
NW = NC * NS
CH = 80
RN = 80
SUP = 25


def _rr_rows(s, n, copy_one):
    n_chunks = n // RN
    iters = (n_chunks + NS - 1) // NS

    def body(t, carry):
        cid = s + t * NS

        @pl.when(cid < n_chunks)
        def _():
            copy_one(cid * RN)

        return carry

    lax.fori_loop(0, iters, body, 0)


def _segsum_sc(x, src, dst, zeros):
    n, d = x.shape
    e = src.shape[0]
    e_per_w = e // NW
    n_chunks = e_per_w // CH
    mesh = plsc.VectorSubcoreMesh(core_axis_name="c", subcore_axis_name="s")

    @functools.partial(
        pl.kernel,
        out_type=jax.ShapeDtypeStruct((NC, n, d), jnp.float32),
        mesh=mesh,
        scratch_types=[
            pltpu.VMEM((SUP * CH,), jnp.int32),
            pltpu.VMEM((SUP * CH,), jnp.int32),
            pltpu.VMEM((CH, d), jnp.float32),
            pltpu.VMEM((CH, d), jnp.float32),
            pltpu.VMEM_SHARED((n, d), jnp.float32),
            pltpu.SemaphoreType.DMA,
            pltpu.SemaphoreType.DMA,
        ],
    )
    def k(x_hbm, src_hbm, dst_hbm, zeros_hbm, out_hbm,
          sidx, didx, rows0, rows1, acc, g0, g1):
        c = lax.axis_index("c")
        s = lax.axis_index("s")
        w = c * NS + s
        _rr_rows(s, n, lambda r0: pltpu.sync_copy(
            zeros_hbm.at[pl.ds(r0, RN)], acc.at[pl.ds(r0, RN)]))
        plsc.subcore_barrier()

        def start_g(j, rows, sem):
            pltpu.async_copy(x_hbm.at[sidx.at[pl.ds(j * CH, CH)]], rows, sem)

        def wait_g(rows, sem):
            pltpu.make_async_copy(zeros_hbm.at[pl.ds(0, CH)], rows, sem).wait()

        def scatter(j, rows):
            pltpu.sync_copy(rows, acc.at[didx.at[pl.ds(j * CH, CH)]], add=True)

        def super_body(u, carry):
            off = w * e_per_w + u * (SUP * CH)
            pltpu.sync_copy(src_hbm.at[pl.ds(off, SUP * CH)], sidx)
            pltpu.sync_copy(dst_hbm.at[pl.ds(off, SUP * CH)], didx)
            start_g(0, rows0, g0)
            if SUP > 1:
                start_g(1, rows1, g1)

            def body(t, carry2):
                a = 2 * t
                wait_g(rows0, g0)
                scatter(a, rows0)

                @pl.when(a + 2 < SUP)
                def _():
                    start_g(a + 2, rows0, g0)

                wait_g(rows1, g1)
                scatter(a + 1, rows1)

                @pl.when(a + 3 < SUP)
                def _():
                    start_g(a + 3, rows1, g1)

                return carry2

            lax.fori_loop(0, SUP // 2, body, 0)
            if SUP % 2:
                wait_g(rows0, g0)
                scatter(SUP - 1, rows0)
            return carry

        lax.fori_loop(0, n_chunks // SUP, super_body, 0)
        plsc.subcore_barrier()
        _rr_rows(s, n, lambda r0: pltpu.sync_copy(
            acc.at[pl.ds(r0, RN)], out_hbm.at[c, pl.ds(r0, RN)]))

    return k(x, src, dst, zeros)


def _count_sc(dst, zeros, n, d):
    e = dst.shape[0]
    e_per_w = e // NW
    n_chunks = e_per_w // CH
    depth = 4
    mesh = plsc.VectorSubcoreMesh(core_axis_name="c", subcore_axis_name="s")

    @functools.partial(
        pl.kernel,
        out_type=jax.ShapeDtypeStruct((NC, n, d), jnp.float32),
        mesh=mesh,
        scratch_types=[
            pltpu.VMEM((SUP * CH,), jnp.int32),
            pltpu.VMEM((CH, d), jnp.float32),
            pltpu.VMEM_SHARED((n, d), jnp.float32),
            pltpu.SemaphoreType.DMA,
        ],
    )
    def k(dst_hbm, zeros_hbm, out_hbm, didx, ones_v, acc, ssem):
        c = lax.axis_index("c")
        s = lax.axis_index("s")
        w = c * NS + s

        def fill(i, carry):
            for kk in range(d // 16):
                ones_v[i, pl.ds(kk * 16, 16)] = jnp.ones((16,), jnp.float32)
            return carry

        lax.fori_loop(0, CH, fill, 0)
        _rr_rows(s, n, lambda r0: pltpu.sync_copy(
            zeros_hbm.at[pl.ds(r0, RN)], acc.at[pl.ds(r0, RN)]))
        plsc.subcore_barrier()

        def wait_one():
            pltpu.make_async_copy(zeros_hbm.at[pl.ds(0, CH)], ones_v, ssem).wait()

        def super_body(u, carry):
            off = w * e_per_w + u * (SUP * CH)
            pltpu.sync_copy(dst_hbm.at[pl.ds(off, SUP * CH)], didx)

            def body(j, carry2):
                pltpu.async_copy(ones_v, acc.at[didx.at[pl.ds(j * CH, CH)]],
                                 ssem, add=True)

                @pl.when(j >= depth)
                def _():
                    wait_one()

                return carry2

            lax.fori_loop(0, SUP, body, 0)
            for _ in range(min(depth, SUP)):
                wait_one()
            return carry

        lax.fori_loop(0, n_chunks // SUP, super_body, 0)
        plsc.subcore_barrier()
        _rr_rows(s, n, lambda r0: pltpu.sync_copy(
            acc.at[pl.ds(r0, RN)], out_hbm.at[c, pl.ds(r0, RN)]))

    return k(dst, zeros)


def _dense_tc(aggp, cntp, h, wl_t, bl, wr_t, relu):
    n, d = h.shape
    bn = 1000

    def body(ap_ref, cp_ref, h_ref, wl_ref, bl_ref, wr_ref, o_ref):
        agg = ap_ref[0] + ap_ref[1]
        cnt = cp_ref[0, :, 0:1] + cp_ref[1, :, 0:1]
        mean = agg / jnp.maximum(cnt, 1.0)
        y = (jnp.dot(mean, wl_ref[...], preferred_element_type=jnp.float32,
                     precision=lax.Precision.HIGHEST)
             + bl_ref[...]
             + jnp.dot(h_ref[...], wr_ref[...], preferred_element_type=jnp.float32,
                       precision=lax.Precision.HIGHEST))
        if relu:
            y = jnp.maximum(y, 0.0)
        o_ref[...] = y

    return pl.pallas_call(
        body,
        out_shape=jax.ShapeDtypeStruct((n, d), jnp.float32),
        grid=(n // bn,),
        in_specs=[
            pl.BlockSpec((NC, bn, d), lambda i: (0, i, 0)),
            pl.BlockSpec((NC, bn, d), lambda i: (0, i, 0)),
            pl.BlockSpec((bn, d), lambda i: (i, 0)),
            pl.BlockSpec((d, d), lambda i: (0, 0)),
            pl.BlockSpec((1, d), lambda i: (0, 0)),
            pl.BlockSpec((d, d), lambda i: (0, 0)),
        ],
        out_specs=pl.BlockSpec((bn, d), lambda i: (i, 0)),
    )(aggp, cntp, h, wl_t, bl, wr_t)


def kernel(x, edge_index, W1l, b1, W1r, W2l, b2, W2r, W3l, b3, W3r):
    n, d = x.shape
    ei = edge_index.astype(jnp.int32)
    src, dst = ei[0], ei[1]
    zeros = jnp.zeros((n, d), jnp.float32)

    cntp = _count_sc(dst, zeros, n, d)

    h = x
    for wl, bl, wr, relu in (
        (W1l, b1, W1r, True),
        (W2l, b2, W2r, True),
        (W3l, b3, W3r, False),
    ):
        aggp = _segsum_sc(h, src, dst, zeros)
        h = _dense_tc(aggp, cntp, h, wl.T, bl.reshape(1, d), wr.T, relu)
    return h
